# baseline (speedup 1.0000x reference)
import functools

import jax
import jax.numpy as jnp
from jax import lax
from jax.experimental import pallas as pl
from jax.experimental.pallas import tpu as pltpu
from jax.experimental.pallas import tpu_sc as plsc

K = 256
B = 128
N = 32768
L = 16
NB = 2048
NCH = NB // L
NGRP = NCH // L
NVEC = N // L
NWORKERS = 32
ROWS_PER_W = B // NWORKERS
U = 16

NPASS = 3
SH_P = (31, 20, 9)
SH_D = (20, 9, 0)
DM = (2047, 2047, 511)
DBITS = (11, 11, 9)


def _sel(p, consts):
    v = jnp.int32(consts[-1])
    for q in range(len(consts) - 2, -1, -1):
        v = jnp.where(p == q, jnp.int32(consts[q]), v)
    return v


def _topk_body(x_hbm, out_hbm, b0, b1, b2, hist_v, scr_v,
               si0, si1, si2, so0, so1, so2):
    bufs = (b0, b1, b2)
    isems = (si0, si1, si2)
    osems = (so0, so1, so2)

    cid = lax.axis_index("c")
    sid = lax.axis_index("s")
    wid = sid * 2 + cid
    base = wid * ROWS_PER_W

    lane = lax.broadcasted_iota(jnp.int32, (L,), 0)
    lane16 = lane * L
    ones = jnp.ones((L,), jnp.int32)
    zeros = jnp.zeros((L,), jnp.int32)
    mask31 = jnp.int32(0x7FFFFFFF)

    def extract(vec, i):
        return jnp.sum(jnp.where(lane == i, vec, 0))

    def zero_hist():
        def zbody(i, _):
            for u in range(U):
                hist_v[pl.ds((i * U + u) * L, L)] = zeros
            return 0
        lax.fori_loop(0, NB // L // U, zbody, 0)

    def hist_pass1(row_v):
        def body(i, _):
            vs = [row_v[pl.ds((i * U + u) * L, L)] for u in range(U)]
            dd = [lax.bitwise_and(lax.shift_right_logical(v, 20), 2047)
                  for v in vs]
            for u in range(U):
                plsc.addupdate_scatter(hist_v, [dd[u]], ones)
            return 0
        lax.fori_loop(0, NVEC // U, body, 0)

    def hist_pass2(row_v, p1):
        def body(i, _):
            vs = [row_v[pl.ds((i * U + u) * L, L)] for u in range(U)]
            dd = [lax.bitwise_and(lax.shift_right_logical(v, 9), 2047)
                  for v in vs]
            mm = [lax.bitwise_and(lax.shift_right_logical(v, 20), 2047) == p1
                  for v in vs]
            for u in range(U):
                plsc.addupdate_scatter(hist_v, [dd[u]], ones, mask=mm[u])
            return 0
        lax.fori_loop(0, NVEC // U, body, 0)

    def hist_pass3(row_v, p12):
        def body(i, _):
            vs = [row_v[pl.ds((i * U + u) * L, L)] for u in range(U)]
            uu = [lax.shift_left(v, 1) for v in vs]
            dd = [lax.bitwise_and(v, 511) for v in vs]
            mm = [lax.shift_right_logical(u, 10) == p12 for u in uu]
            for u in range(U):
                plsc.addupdate_scatter(hist_v, [dd[u]], ones, mask=mm[u])
            return 0
        lax.fori_loop(0, NVEC // U, body, 0)

    def scan_hist(kin):
        def sbody(i, _):
            cs = [plsc.cumsum(lax.rev(hist_v[pl.ds((i * 8 + u) * L, L)], (0,)))
                  for u in range(8)]
            for u in range(8):
                scr_v[pl.ds((i * 8 + u) * L, L)] = cs[u]
            return 0
        lax.fori_loop(0, NCH // 8, sbody, 0)
        tots = [plsc.load_gather(scr_v, [lane16 + (g * 256 + (L - 1))])
                for g in range(NGRP)]
        csg = [plsc.cumsum(lax.rev(t, (0,))) for t in tots]
        gts = [jnp.max(c) for c in csg]
        cum = jnp.int32(0)
        found = jnp.int32(0)
        gstar = jnp.int32(0)
        need_g = jnp.int32(0)
        cs_g = zeros
        for g in range(NGRP - 1, -1, -1):
            hit = jnp.logical_and(found == 0, cum + gts[g] >= kin)
            hb = (zeros + hit.astype(jnp.int32)) == 1
            gstar = jnp.where(hit, g, gstar)
            need_g = jnp.where(hit, kin - cum, need_g)
            cs_g = jnp.where(hb, csg[g], cs_g)
            found = jnp.where(hit, 1, found)
            cum = cum + gts[g]
        q1 = jnp.max(plsc.all_reduce_ffs(cs_g >= need_g))
        tstar = gstar * L + (L - 1) - q1
        need_c = need_g - extract(cs_g, q1 - 1)
        cs_star = scr_v[pl.ds(tstar * L, L)]
        q2 = jnp.max(plsc.all_reduce_ffs(cs_star >= need_c))
        digit = tstar * L + (L - 1) - q2
        above = extract(cs_star, q2 - 1)
        kin_next = need_c - above
        cnt_b = extract(cs_star, q2) - above
        return digit, kin_next, cnt_b

    def compute_threshold(row_v):
        kin = jnp.int32(K)
        zero_hist()
        hist_pass1(row_v)
        p1, kin, _ = scan_hist(kin)
        zero_hist()
        hist_pass2(row_v, p1)
        d2, kin, _ = scan_hist(kin)
        p12 = lax.bitwise_or(lax.shift_left(p1, 11), d2)
        zero_hist()
        hist_pass3(row_v, p12)
        d3, kin3, cnt3 = scan_hist(kin)
        T = lax.bitwise_or(lax.shift_left(p12, 9), d3)
        return T, cnt3 - kin3

    def output_pass(row_v, T):
        def obody(i, _):
            vs = [row_v[pl.ds((i * U + u) * L, L)] for u in range(U)]
            aa = [lax.bitwise_and(v, mask31) for v in vs]
            oo = [jnp.where(a >= T, a, 0) for a in aa]
            for u in range(U):
                row_v[pl.ds((i * U + u) * L, L)] = oo[u]
            return 0
        lax.fori_loop(0, NVEC // U, obody, 0)

    def in_copy(j, buf):
        return pltpu.make_async_copy(x_hbm.at[base + j], buf, isems[j % 3])

    def out_copy(j, buf):
        return pltpu.make_async_copy(buf, out_hbm.at[base + j], osems[j % 3])

    for m in range(3):
        in_copy(m, bufs[m]).start()

    def fix_ties(row_v, T, extra):
        def cond(c):
            i, z = c
            return jnp.logical_and(i < NVEC, z < extra)

        def bodyw(c):
            i, z = c
            vi = NVEC - 1 - i
            o = row_v[pl.ds(vi * L, L)]
            mi = (o == T).astype(jnp.int32)
            rcs = lax.rev(plsc.cumsum(lax.rev(mi, (0,))), (0,))
            zm = jnp.logical_and(mi == 1, rcs + z <= extra)
            row_v[pl.ds(vi * L, L)] = jnp.where(zm, 0, o)
            return (i + 1, z + jnp.max(rcs))
        lax.while_loop(cond, bodyw, (jnp.int32(0), jnp.int32(0)))

    for j in range(ROWS_PER_W):
        bj = bufs[j % 3]
        in_copy(j, bj).wait()
        T, extra = compute_threshold(bj)
        if j == 1:
            out_copy(0, bufs[0]).wait()
            in_copy(3, bufs[0]).start()
        output_pass(bj, T)

        @pl.when(extra > 0)
        def _():
            fix_ties(bj, T, extra)
        out_copy(j, bj).start()

    for j in (1, 2, 3):
        out_copy(j, bufs[j % 3]).wait()


@jax.jit
def _topk_mask(bits):
    mesh = plsc.VectorSubcoreMesh(core_axis_name="c", subcore_axis_name="s")
    f = functools.partial(
        pl.kernel,
        out_type=jax.ShapeDtypeStruct((B, N), jnp.int32),
        mesh=mesh,
        scratch_types=[
            pltpu.VMEM((N,), jnp.int32),
            pltpu.VMEM((N,), jnp.int32),
            pltpu.VMEM((N,), jnp.int32),
            pltpu.VMEM((NB,), jnp.int32),
            pltpu.VMEM((NB,), jnp.int32),
            pltpu.SemaphoreType.DMA,
            pltpu.SemaphoreType.DMA,
            pltpu.SemaphoreType.DMA,
            pltpu.SemaphoreType.DMA,
            pltpu.SemaphoreType.DMA,
            pltpu.SemaphoreType.DMA,
        ],
        compiler_params=pltpu.CompilerParams(needs_layout_passes=False),
    )(_topk_body)
    return f(bits)


def kernel(inputs):
    bits = lax.bitcast_convert_type(inputs, jnp.int32)
    out_bits = _topk_mask(bits)
    return lax.bitcast_convert_type(out_bits, jnp.float32)


# speedup vs baseline: 1.1016x; 1.1016x over previous
import functools

import jax
import jax.numpy as jnp
from jax import lax
from jax.experimental import pallas as pl
from jax.experimental.pallas import tpu as pltpu
from jax.experimental.pallas import tpu_sc as plsc

K = 256
B = 128
N = 32768
L = 16
NB = 2048
NCH = NB // L
NGRP = NCH // L
NVEC = N // L
NWORKERS = 32
ROWS_PER_W = B // NWORKERS
U = 16

NPASS = 3
SH_P = (31, 20, 9)
SH_D = (20, 9, 0)
DM = (2047, 2047, 511)
DBITS = (11, 11, 9)


def _sel(p, consts):
    v = jnp.int32(consts[-1])
    for q in range(len(consts) - 2, -1, -1):
        v = jnp.where(p == q, jnp.int32(consts[q]), v)
    return v


def _topk_body(x_hbm, out_hbm, b0, b1, b2, hist_v, scr_v,
               si0, si1, si2, so0, so1, so2):
    bufs = (b0, b1, b2)
    isems = (si0, si1, si2)
    osems = (so0, so1, so2)

    cid = lax.axis_index("c")
    sid = lax.axis_index("s")
    wid = sid * 2 + cid
    base = wid * ROWS_PER_W

    lane = lax.broadcasted_iota(jnp.int32, (L,), 0)
    lane16 = lane * L
    ones = jnp.ones((L,), jnp.int32)
    zeros = jnp.zeros((L,), jnp.int32)
    mask31 = jnp.int32(0x7FFFFFFF)

    def extract(vec, i):
        return jnp.sum(jnp.where(lane == i, vec, 0))

    def zero_hist():
        def zbody(i, _):
            for u in range(U):
                hist_v[pl.ds((i * U + u) * L, L)] = zeros
            return 0
        lax.fori_loop(0, NB // L // U, zbody, 0)

    def hist_pass1(row_v):
        def body(i, _):
            vs = [row_v[pl.ds((i * U + u) * L, L)] for u in range(U)]
            dd = [lax.bitwise_and(lax.shift_right_logical(v, 20), 2047)
                  for v in vs]
            for u in range(U):
                plsc.addupdate_scatter(hist_v, [dd[u]], ones)
            return 0
        lax.fori_loop(0, NVEC // U, body, 0)

    def hist_pass2(row_v, p1):
        def body(i, _):
            vs = [row_v[pl.ds((i * U + u) * L, L)] for u in range(U)]
            dd = [lax.bitwise_and(lax.shift_right_logical(v, 9), 2047)
                  for v in vs]
            mm = [lax.bitwise_and(lax.shift_right_logical(v, 20), 2047) == p1
                  for v in vs]
            for u in range(U):
                plsc.addupdate_scatter(hist_v, [dd[u]], ones, mask=mm[u])
            return 0
        lax.fori_loop(0, NVEC // U, body, 0)

    def hist_pass3(row_v, p12):
        def body(i, _):
            vs = [row_v[pl.ds((i * U + u) * L, L)] for u in range(U)]
            uu = [lax.shift_left(v, 1) for v in vs]
            dd = [lax.bitwise_and(v, 511) for v in vs]
            mm = [lax.shift_right_logical(u, 10) == p12 for u in uu]
            for u in range(U):
                plsc.addupdate_scatter(hist_v, [dd[u]], ones, mask=mm[u])
            return 0
        lax.fori_loop(0, NVEC // U, body, 0)

    def scan_hist(kin):
        def sbody(i, _):
            cs = [plsc.cumsum(lax.rev(hist_v[pl.ds((i * 8 + u) * L, L)], (0,)))
                  for u in range(8)]
            for u in range(8):
                scr_v[pl.ds((i * 8 + u) * L, L)] = cs[u]
            return 0
        lax.fori_loop(0, NCH // 8, sbody, 0)
        tots = [plsc.load_gather(scr_v, [lane16 + (g * 256 + (L - 1))])
                for g in range(NGRP)]
        csg = [plsc.cumsum(lax.rev(t, (0,))) for t in tots]
        gts = [jnp.max(c) for c in csg]
        cum = jnp.int32(0)
        found = jnp.int32(0)
        gstar = jnp.int32(0)
        need_g = jnp.int32(0)
        cs_g = zeros
        for g in range(NGRP - 1, -1, -1):
            hit = jnp.logical_and(found == 0, cum + gts[g] >= kin)
            hb = (zeros + hit.astype(jnp.int32)) == 1
            gstar = jnp.where(hit, g, gstar)
            need_g = jnp.where(hit, kin - cum, need_g)
            cs_g = jnp.where(hb, csg[g], cs_g)
            found = jnp.where(hit, 1, found)
            cum = cum + gts[g]
        q1 = jnp.max(plsc.all_reduce_ffs(cs_g >= need_g))
        tstar = gstar * L + (L - 1) - q1
        need_c = need_g - extract(cs_g, q1 - 1)
        cs_star = scr_v[pl.ds(tstar * L, L)]
        q2 = jnp.max(plsc.all_reduce_ffs(cs_star >= need_c))
        digit = tstar * L + (L - 1) - q2
        above = extract(cs_star, q2 - 1)
        kin_next = need_c - above
        cnt_b = extract(cs_star, q2) - above
        return digit, kin_next, cnt_b

    def compute_threshold(row_v):
        kin = jnp.int32(K)
        zero_hist()
        hist_pass1(row_v)
        p1, kin, _ = scan_hist(kin)
        zero_hist()
        hist_pass2(row_v, p1)
        d2, kin, _ = scan_hist(kin)
        p12 = lax.bitwise_or(lax.shift_left(p1, 11), d2)
        zero_hist()
        hist_pass3(row_v, p12)
        d3, kin3, cnt3 = scan_hist(kin)
        T = lax.bitwise_or(lax.shift_left(p12, 9), d3)
        return T, cnt3 - kin3

    def output_pass(row_v, T):
        def obody(i, _):
            vs = [row_v[pl.ds((i * U + u) * L, L)] for u in range(U)]
            aa = [lax.bitwise_and(v, mask31) for v in vs]
            oo = [jnp.where(a >= T, a, 0) for a in aa]
            for u in range(U):
                row_v[pl.ds((i * U + u) * L, L)] = oo[u]
            return 0
        lax.fori_loop(0, NVEC // U, obody, 0)

    def in_copy(j, buf):
        return pltpu.make_async_copy(x_hbm.at[base + j], buf, isems[j % 3])

    def out_copy(j, buf):
        return pltpu.make_async_copy(buf, out_hbm.at[base + j], osems[j % 3])

    for m in range(3):
        in_copy(m, bufs[m]).start()

    def fix_ties(row_v, T, extra):
        def fbody(i, z):
            os = [row_v[pl.ds((NVEC - (i * U + u + 1)) * L, L)]
                  for u in range(U)]
            mis = [(o == T).astype(jnp.int32) for o in os]
            rcss = [lax.rev(plsc.cumsum(lax.rev(mi, (0,))), (0,))
                    for mi in mis]
            ts = [jnp.max(rcs) for rcs in rcss]
            zs = [z]
            for u in range(1, U):
                zs.append(zs[u - 1] + ts[u - 1])
            for u in range(U):
                zm = jnp.logical_and(mis[u] == 1, rcss[u] + zs[u] <= extra)
                row_v[pl.ds((NVEC - (i * U + u + 1)) * L, L)] = (
                    jnp.where(zm, 0, os[u]))
            return zs[U - 1] + ts[U - 1]
        lax.fori_loop(0, NVEC // U, fbody, jnp.int32(0))

    for j in range(ROWS_PER_W):
        bj = bufs[j % 3]
        in_copy(j, bj).wait()
        T, extra = compute_threshold(bj)
        if j == 1:
            out_copy(0, bufs[0]).wait()
            in_copy(3, bufs[0]).start()
        output_pass(bj, T)

        @pl.when(extra > 0)
        def _():
            fix_ties(bj, T, extra)
        out_copy(j, bj).start()

    for j in (1, 2, 3):
        out_copy(j, bufs[j % 3]).wait()


@jax.jit
def _topk_mask(bits):
    mesh = plsc.VectorSubcoreMesh(core_axis_name="c", subcore_axis_name="s")
    f = functools.partial(
        pl.kernel,
        out_type=jax.ShapeDtypeStruct((B, N), jnp.int32),
        mesh=mesh,
        scratch_types=[
            pltpu.VMEM((N,), jnp.int32),
            pltpu.VMEM((N,), jnp.int32),
            pltpu.VMEM((N,), jnp.int32),
            pltpu.VMEM((NB,), jnp.int32),
            pltpu.VMEM((NB,), jnp.int32),
            pltpu.SemaphoreType.DMA,
            pltpu.SemaphoreType.DMA,
            pltpu.SemaphoreType.DMA,
            pltpu.SemaphoreType.DMA,
            pltpu.SemaphoreType.DMA,
            pltpu.SemaphoreType.DMA,
        ],
        compiler_params=pltpu.CompilerParams(needs_layout_passes=False),
    )(_topk_body)
    return f(bits)


def kernel(inputs):
    bits = lax.bitcast_convert_type(inputs, jnp.int32)
    out_bits = _topk_mask(bits)
    return lax.bitcast_convert_type(out_bits, jnp.float32)


# speedup vs baseline: 1.1440x; 1.0385x over previous
import functools

import jax
import jax.numpy as jnp
from jax import lax
from jax.experimental import pallas as pl
from jax.experimental.pallas import tpu as pltpu
from jax.experimental.pallas import tpu_sc as plsc

K = 256
B = 128
N = 32768
L = 16
NB = 2048
NCH = NB // L
NGRP = NCH // L
NVEC = N // L
NWORKERS = 32
ROWS_PER_W = B // NWORKERS
U = 16

NPASS = 3
SH_P = (31, 20, 9)
SH_D = (20, 9, 0)
DM = (2047, 2047, 511)
DBITS = (11, 11, 9)


def _sel(p, consts):
    v = jnp.int32(consts[-1])
    for q in range(len(consts) - 2, -1, -1):
        v = jnp.where(p == q, jnp.int32(consts[q]), v)
    return v


def _topk_body(x_hbm, out_hbm, b0, b1, b2, hist_v, scr_v,
               si0, si1, si2, so0, so1, so2):
    bufs = (b0, b1, b2)
    isems = (si0, si1, si2)
    osems = (so0, so1, so2)

    cid = lax.axis_index("c")
    sid = lax.axis_index("s")
    wid = sid * 2 + cid
    base = wid * ROWS_PER_W

    lane = lax.broadcasted_iota(jnp.int32, (L,), 0)
    lane16 = lane * L
    ones = jnp.ones((L,), jnp.int32)
    zeros = jnp.zeros((L,), jnp.int32)
    mask31 = jnp.int32(0x7FFFFFFF)

    def extract(vec, i):
        return jnp.sum(jnp.where(lane == i, vec, 0))

    def zero_hist():
        def zbody(i, _):
            for u in range(U):
                hist_v[pl.ds((i * U + u) * L, L)] = zeros
            return 0
        lax.fori_loop(0, NB // L // U, zbody, 0)

    def hist_pass1(row_v):
        def body(i, _):
            vs = [row_v[pl.ds((i * U + u) * L, L)] for u in range(U)]
            dd = [lax.bitwise_and(lax.shift_right_logical(v, 20), 2047)
                  for v in vs]
            for u in range(U):
                plsc.addupdate_scatter(hist_v, [dd[u]], ones)
            return 0
        lax.fori_loop(0, NVEC // U, body, 0)

    def hist_pass2(row_v, p1):
        def body(i, _):
            vs = [row_v[pl.ds((i * U + u) * L, L)] for u in range(U)]
            dd = [lax.bitwise_and(lax.shift_right_logical(v, 9), 2047)
                  for v in vs]
            mm = [lax.bitwise_and(lax.shift_right_logical(v, 20), 2047) == p1
                  for v in vs]
            for u in range(U):
                plsc.addupdate_scatter(hist_v, [dd[u]], ones, mask=mm[u])
            return 0
        lax.fori_loop(0, NVEC // U, body, 0)

    def hist_pass3(row_v, p12):
        def body(i, _):
            vs = [row_v[pl.ds((i * U + u) * L, L)] for u in range(U)]
            uu = [lax.shift_left(v, 1) for v in vs]
            dd = [lax.bitwise_and(v, 511) for v in vs]
            mm = [lax.shift_right_logical(u, 10) == p12 for u in uu]
            for u in range(U):
                plsc.addupdate_scatter(hist_v, [dd[u]], ones, mask=mm[u])
            return 0
        lax.fori_loop(0, NVEC // U, body, 0)

    def scan_hist(kin):
        def sbody(i, _):
            cs = [plsc.cumsum(lax.rev(hist_v[pl.ds((i * 8 + u) * L, L)], (0,)))
                  for u in range(8)]
            for u in range(8):
                scr_v[pl.ds((i * 8 + u) * L, L)] = cs[u]
            return 0
        lax.fori_loop(0, NCH // 8, sbody, 0)
        tots = [plsc.load_gather(scr_v, [lane16 + (g * 256 + (L - 1))])
                for g in range(NGRP)]
        csg = [plsc.cumsum(lax.rev(t, (0,))) for t in tots]
        gts = [jnp.max(c) for c in csg]
        cum = jnp.int32(0)
        found = jnp.int32(0)
        gstar = jnp.int32(0)
        need_g = jnp.int32(0)
        cs_g = zeros
        for g in range(NGRP - 1, -1, -1):
            hit = jnp.logical_and(found == 0, cum + gts[g] >= kin)
            hb = (zeros + hit.astype(jnp.int32)) == 1
            gstar = jnp.where(hit, g, gstar)
            need_g = jnp.where(hit, kin - cum, need_g)
            cs_g = jnp.where(hb, csg[g], cs_g)
            found = jnp.where(hit, 1, found)
            cum = cum + gts[g]
        q1 = jnp.max(plsc.all_reduce_ffs(cs_g >= need_g))
        tstar = gstar * L + (L - 1) - q1
        need_c = need_g - extract(cs_g, q1 - 1)
        cs_star = scr_v[pl.ds(tstar * L, L)]
        q2 = jnp.max(plsc.all_reduce_ffs(cs_star >= need_c))
        digit = tstar * L + (L - 1) - q2
        above = extract(cs_star, q2 - 1)
        kin_next = need_c - above
        cnt_b = extract(cs_star, q2) - above
        return digit, kin_next, cnt_b

    def compute_threshold(row_v):
        kin = jnp.int32(K)
        zero_hist()
        hist_pass1(row_v)
        p1, kin, _ = scan_hist(kin)
        zero_hist()
        hist_pass2(row_v, p1)
        d2, kin, _ = scan_hist(kin)
        p12 = lax.bitwise_or(lax.shift_left(p1, 11), d2)
        zero_hist()
        hist_pass3(row_v, p12)
        d3, kin3, cnt3 = scan_hist(kin)
        T = lax.bitwise_or(lax.shift_left(p12, 9), d3)
        return T, cnt3 - kin3

    def output_pass(row_v, T):
        def obody(i, _):
            vs = [row_v[pl.ds((i * U + u) * L, L)] for u in range(U)]
            aa = [lax.bitwise_and(v, mask31) for v in vs]
            oo = [jnp.where(a >= T, a, 0) for a in aa]
            for u in range(U):
                row_v[pl.ds((i * U + u) * L, L)] = oo[u]
            return 0
        lax.fori_loop(0, NVEC // U, obody, 0)

    def in_copy(j, buf):
        return pltpu.make_async_copy(x_hbm.at[base + j], buf, isems[j % 3])

    def out_copy(j, buf):
        return pltpu.make_async_copy(buf, out_hbm.at[base + j], osems[j % 3])

    for m in range(3):
        in_copy(m, bufs[m]).start()

    def fix_ties(row_v, T, extra):
        def cond(c):
            i, z = c
            return jnp.logical_and(i < NVEC // U, z < extra)

        def fbody(c):
            i, z = c
            os = [row_v[pl.ds((NVEC - (i * U + u + 1)) * L, L)]
                  for u in range(U)]
            mis = [(o == T).astype(jnp.int32) for o in os]
            rcss = [lax.rev(plsc.cumsum(lax.rev(mi, (0,))), (0,))
                    for mi in mis]
            ts = [jnp.max(rcs) for rcs in rcss]
            zs = [z]
            for u in range(1, U):
                zs.append(zs[u - 1] + ts[u - 1])
            for u in range(U):
                zm = jnp.logical_and(mis[u] == 1, rcss[u] + zs[u] <= extra)
                row_v[pl.ds((NVEC - (i * U + u + 1)) * L, L)] = (
                    jnp.where(zm, 0, os[u]))
            return (i + 1, zs[U - 1] + ts[U - 1])
        lax.while_loop(cond, fbody, (jnp.int32(0), jnp.int32(0)))

    for j in range(ROWS_PER_W):
        bj = bufs[j % 3]
        in_copy(j, bj).wait()
        T, extra = compute_threshold(bj)
        if j == 1:
            out_copy(0, bufs[0]).wait()
            in_copy(3, bufs[0]).start()
        output_pass(bj, T)

        @pl.when(extra > 0)
        def _():
            fix_ties(bj, T, extra)
        out_copy(j, bj).start()

    for j in (1, 2, 3):
        out_copy(j, bufs[j % 3]).wait()


@jax.jit
def _topk_mask(bits):
    mesh = plsc.VectorSubcoreMesh(core_axis_name="c", subcore_axis_name="s")
    f = functools.partial(
        pl.kernel,
        out_type=jax.ShapeDtypeStruct((B, N), jnp.int32),
        mesh=mesh,
        scratch_types=[
            pltpu.VMEM((N,), jnp.int32),
            pltpu.VMEM((N,), jnp.int32),
            pltpu.VMEM((N,), jnp.int32),
            pltpu.VMEM((NB,), jnp.int32),
            pltpu.VMEM((NB,), jnp.int32),
            pltpu.SemaphoreType.DMA,
            pltpu.SemaphoreType.DMA,
            pltpu.SemaphoreType.DMA,
            pltpu.SemaphoreType.DMA,
            pltpu.SemaphoreType.DMA,
            pltpu.SemaphoreType.DMA,
        ],
        compiler_params=pltpu.CompilerParams(needs_layout_passes=False),
    )(_topk_body)
    return f(bits)


def kernel(inputs):
    bits = lax.bitcast_convert_type(inputs, jnp.int32)
    out_bits = _topk_mask(bits)
    return lax.bitcast_convert_type(out_bits, jnp.float32)


# speedup vs baseline: 1.1588x; 1.0129x over previous
import functools

import jax
import jax.numpy as jnp
from jax import lax
from jax.experimental import pallas as pl
from jax.experimental.pallas import tpu as pltpu
from jax.experimental.pallas import tpu_sc as plsc

K = 256
B = 128
N = 32768
L = 16
NB = 2048
NCH = NB // L
NGRP = NCH // L
NVEC = N // L
NWORKERS = 32
ROWS_PER_W = B // NWORKERS
U = 16

NPASS = 3
SH_P = (31, 20, 9)
SH_D = (20, 9, 0)
DM = (2047, 2047, 511)
DBITS = (11, 11, 9)


def _sel(p, consts):
    v = jnp.int32(consts[-1])
    for q in range(len(consts) - 2, -1, -1):
        v = jnp.where(p == q, jnp.int32(consts[q]), v)
    return v


def _topk_body(x_hbm, out_hbm, b0, b1, b2, hist_v, scr_v,
               si0, si1, si2, si3, so0, so1, so2):
    bufs = (b0, b1, b2)
    isems = (si0, si1, si2)
    osems = (so0, so1, so2)

    cid = lax.axis_index("c")
    sid = lax.axis_index("s")
    wid = sid * 2 + cid
    base = wid * ROWS_PER_W

    lane = lax.broadcasted_iota(jnp.int32, (L,), 0)
    lane16 = lane * L
    ones = jnp.ones((L,), jnp.int32)
    zeros = jnp.zeros((L,), jnp.int32)
    mask31 = jnp.int32(0x7FFFFFFF)

    def extract(vec, i):
        return jnp.sum(jnp.where(lane == i, vec, 0))

    def zero_hist():
        def zbody(i, _):
            for u in range(U):
                hist_v[pl.ds((i * U + u) * L, L)] = zeros
            return 0
        lax.fori_loop(0, NB // L // U, zbody, 0)

    def hist_pass1(row_v, v0=0, nvec=NVEC):
        def body(i, _):
            vs = [row_v[pl.ds((v0 + i * U + u) * L, L)] for u in range(U)]
            dd = [lax.bitwise_and(lax.shift_right_logical(v, 20), 2047)
                  for v in vs]
            for u in range(U):
                plsc.addupdate_scatter(hist_v, [dd[u]], ones)
            return 0
        lax.fori_loop(0, nvec // U, body, 0)

    def hist_pass2(row_v, p1):
        def body(i, _):
            vs = [row_v[pl.ds((i * U + u) * L, L)] for u in range(U)]
            dd = [lax.bitwise_and(lax.shift_right_logical(v, 9), 2047)
                  for v in vs]
            mm = [lax.bitwise_and(lax.shift_right_logical(v, 20), 2047) == p1
                  for v in vs]
            for u in range(U):
                plsc.addupdate_scatter(hist_v, [dd[u]], ones, mask=mm[u])
            return 0
        lax.fori_loop(0, NVEC // U, body, 0)

    def hist_pass3(row_v, p12):
        def body(i, _):
            vs = [row_v[pl.ds((i * U + u) * L, L)] for u in range(U)]
            uu = [lax.shift_left(v, 1) for v in vs]
            dd = [lax.bitwise_and(v, 511) for v in vs]
            mm = [lax.shift_right_logical(u, 10) == p12 for u in uu]
            for u in range(U):
                plsc.addupdate_scatter(hist_v, [dd[u]], ones, mask=mm[u])
            return 0
        lax.fori_loop(0, NVEC // U, body, 0)

    def scan_hist(kin):
        def sbody(i, _):
            cs = [plsc.cumsum(lax.rev(hist_v[pl.ds((i * 8 + u) * L, L)], (0,)))
                  for u in range(8)]
            for u in range(8):
                scr_v[pl.ds((i * 8 + u) * L, L)] = cs[u]
            return 0
        lax.fori_loop(0, NCH // 8, sbody, 0)
        tots = [plsc.load_gather(scr_v, [lane16 + (g * 256 + (L - 1))])
                for g in range(NGRP)]
        csg = [plsc.cumsum(lax.rev(t, (0,))) for t in tots]
        gts = [jnp.max(c) for c in csg]
        cum = jnp.int32(0)
        found = jnp.int32(0)
        gstar = jnp.int32(0)
        need_g = jnp.int32(0)
        cs_g = zeros
        for g in range(NGRP - 1, -1, -1):
            hit = jnp.logical_and(found == 0, cum + gts[g] >= kin)
            hb = (zeros + hit.astype(jnp.int32)) == 1
            gstar = jnp.where(hit, g, gstar)
            need_g = jnp.where(hit, kin - cum, need_g)
            cs_g = jnp.where(hb, csg[g], cs_g)
            found = jnp.where(hit, 1, found)
            cum = cum + gts[g]
        q1 = jnp.max(plsc.all_reduce_ffs(cs_g >= need_g))
        tstar = gstar * L + (L - 1) - q1
        need_c = need_g - extract(cs_g, q1 - 1)
        cs_star = scr_v[pl.ds(tstar * L, L)]
        q2 = jnp.max(plsc.all_reduce_ffs(cs_star >= need_c))
        digit = tstar * L + (L - 1) - q2
        above = extract(cs_star, q2 - 1)
        kin_next = need_c - above
        cnt_b = extract(cs_star, q2) - above
        return digit, kin_next, cnt_b

    def compute_threshold(row_v, run_pass1):
        kin = jnp.int32(K)
        zero_hist()
        run_pass1()
        p1, kin, _ = scan_hist(kin)
        zero_hist()
        hist_pass2(row_v, p1)
        d2, kin, _ = scan_hist(kin)
        p12 = lax.bitwise_or(lax.shift_left(p1, 11), d2)
        zero_hist()
        hist_pass3(row_v, p12)
        d3, kin3, cnt3 = scan_hist(kin)
        T = lax.bitwise_or(lax.shift_left(p12, 9), d3)
        return T, cnt3 - kin3

    def output_pass(row_v, T):
        def obody(i, _):
            vs = [row_v[pl.ds((i * U + u) * L, L)] for u in range(U)]
            aa = [lax.bitwise_and(v, mask31) for v in vs]
            oo = [jnp.where(a >= T, a, 0) for a in aa]
            for u in range(U):
                row_v[pl.ds((i * U + u) * L, L)] = oo[u]
            return 0
        lax.fori_loop(0, NVEC // U, obody, 0)

    def in_copy(j, buf):
        return pltpu.make_async_copy(x_hbm.at[base + j], buf, isems[j % 3])

    def out_copy(j, buf):
        return pltpu.make_async_copy(buf, out_hbm.at[base + j], osems[j % 3])

    HV = NVEC // 2
    HN = N // 2

    def in_half(h):
        sem = isems[0] if h == 0 else si3
        return pltpu.make_async_copy(
            x_hbm.at[base, pl.ds(h * HN, HN)],
            bufs[0].at[pl.ds(h * HN, HN)], sem)

    in_half(0).start()
    in_half(1).start()
    for m in (1, 2):
        in_copy(m, bufs[m]).start()

    def fix_ties(row_v, T, extra):
        def cond(c):
            i, z = c
            return jnp.logical_and(i < NVEC // U, z < extra)

        def fbody(c):
            i, z = c
            os = [row_v[pl.ds((NVEC - (i * U + u + 1)) * L, L)]
                  for u in range(U)]
            mis = [(o == T).astype(jnp.int32) for o in os]
            rcss = [lax.rev(plsc.cumsum(lax.rev(mi, (0,))), (0,))
                    for mi in mis]
            ts = [jnp.max(rcs) for rcs in rcss]
            zs = [z]
            for u in range(1, U):
                zs.append(zs[u - 1] + ts[u - 1])
            for u in range(U):
                zm = jnp.logical_and(mis[u] == 1, rcss[u] + zs[u] <= extra)
                row_v[pl.ds((NVEC - (i * U + u + 1)) * L, L)] = (
                    jnp.where(zm, 0, os[u]))
            return (i + 1, zs[U - 1] + ts[U - 1])
        lax.while_loop(cond, fbody, (jnp.int32(0), jnp.int32(0)))

    for j in range(ROWS_PER_W):
        bj = bufs[j % 3]
        if j == 0:
            def run_pass1(b=bj):
                in_half(0).wait()
                hist_pass1(b, 0, HV)
                in_half(1).wait()
                hist_pass1(b, HV, HV)
        else:
            in_copy(j, bj).wait()

            def run_pass1(b=bj):
                hist_pass1(b)
        T, extra = compute_threshold(bj, run_pass1)
        if j == 1:
            out_copy(0, bufs[0]).wait()
            in_copy(3, bufs[0]).start()
        output_pass(bj, T)

        @pl.when(extra > 0)
        def _():
            fix_ties(bj, T, extra)
        out_copy(j, bj).start()

    for j in (1, 2, 3):
        out_copy(j, bufs[j % 3]).wait()


@jax.jit
def _topk_mask(bits):
    mesh = plsc.VectorSubcoreMesh(core_axis_name="c", subcore_axis_name="s")
    f = functools.partial(
        pl.kernel,
        out_type=jax.ShapeDtypeStruct((B, N), jnp.int32),
        mesh=mesh,
        scratch_types=[
            pltpu.VMEM((N,), jnp.int32),
            pltpu.VMEM((N,), jnp.int32),
            pltpu.VMEM((N,), jnp.int32),
            pltpu.VMEM((NB,), jnp.int32),
            pltpu.VMEM((NB,), jnp.int32),
            pltpu.SemaphoreType.DMA,
            pltpu.SemaphoreType.DMA,
            pltpu.SemaphoreType.DMA,
            pltpu.SemaphoreType.DMA,
            pltpu.SemaphoreType.DMA,
            pltpu.SemaphoreType.DMA,
            pltpu.SemaphoreType.DMA,
        ],
        compiler_params=pltpu.CompilerParams(needs_layout_passes=False),
    )(_topk_body)
    return f(bits)


def kernel(inputs):
    bits = lax.bitcast_convert_type(inputs, jnp.int32)
    out_bits = _topk_mask(bits)
    return lax.bitcast_convert_type(out_bits, jnp.float32)


# speedup vs baseline: 1.1597x; 1.0008x over previous
import functools

import jax
import jax.numpy as jnp
from jax import lax
from jax.experimental import pallas as pl
from jax.experimental.pallas import tpu as pltpu
from jax.experimental.pallas import tpu_sc as plsc

K = 256
B = 128
N = 32768
L = 16
NB = 2048
NCH = NB // L
NGRP = NCH // L
NVEC = N // L
NWORKERS = 32
ROWS_PER_W = B // NWORKERS
U = 16

def _topk_body(x_hbm, out_hbm, b0, b1, b2, hist_v, scr_v,
               si0, si1, si2, si3, so0, so1, so2):
    bufs = (b0, b1, b2)
    isems = (si0, si1, si2)
    osems = (so0, so1, so2)

    cid = lax.axis_index("c")
    sid = lax.axis_index("s")
    wid = sid * 2 + cid
    base = wid * ROWS_PER_W

    lane = lax.broadcasted_iota(jnp.int32, (L,), 0)
    lane16 = lane * L
    ones = jnp.ones((L,), jnp.int32)
    zeros = jnp.zeros((L,), jnp.int32)
    mask31 = jnp.int32(0x7FFFFFFF)

    def extract(vec, i):
        return jnp.sum(jnp.where(lane == i, vec, 0))

    def zero_hist():
        def zbody(i, _):
            for u in range(U):
                hist_v[pl.ds((i * U + u) * L, L)] = zeros
            return 0
        lax.fori_loop(0, NB // L // U, zbody, 0)

    def hist_pass1(row_v, v0=0, nvec=NVEC):
        def body(i, _):
            vs = [row_v[pl.ds((v0 + i * U + u) * L, L)] for u in range(U)]
            dd = [lax.bitwise_and(lax.shift_right_logical(v, 20), 2047)
                  for v in vs]
            for u in range(U):
                plsc.addupdate_scatter(hist_v, [dd[u]], ones)
            return 0
        lax.fori_loop(0, nvec // U, body, 0)

    def hist_pass2(row_v, p1):
        def body(i, _):
            vs = [row_v[pl.ds((i * U + u) * L, L)] for u in range(U)]
            dd = [lax.bitwise_and(lax.shift_right_logical(v, 9), 2047)
                  for v in vs]
            mm = [lax.bitwise_and(lax.shift_right_logical(v, 20), 2047) == p1
                  for v in vs]
            for u in range(U):
                plsc.addupdate_scatter(hist_v, [dd[u]], ones, mask=mm[u])
            return 0
        lax.fori_loop(0, NVEC // U, body, 0)

    def hist_pass3(row_v, p12):
        def body(i, _):
            vs = [row_v[pl.ds((i * U + u) * L, L)] for u in range(U)]
            uu = [lax.shift_left(v, 1) for v in vs]
            dd = [lax.bitwise_and(v, 511) for v in vs]
            mm = [lax.shift_right_logical(u, 10) == p12 for u in uu]
            for u in range(U):
                plsc.addupdate_scatter(hist_v, [dd[u]], ones, mask=mm[u])
            return 0
        lax.fori_loop(0, NVEC // U, body, 0)

    def scan_hist(kin):
        def sbody(i, _):
            cs = [plsc.cumsum(lax.rev(hist_v[pl.ds((i * 8 + u) * L, L)], (0,)))
                  for u in range(8)]
            for u in range(8):
                scr_v[pl.ds((i * 8 + u) * L, L)] = cs[u]
            return 0
        lax.fori_loop(0, NCH // 8, sbody, 0)
        tots = [plsc.load_gather(scr_v, [lane16 + (g * 256 + (L - 1))])
                for g in range(NGRP)]
        csg = [plsc.cumsum(lax.rev(t, (0,))) for t in tots]
        gts = [jnp.max(c) for c in csg]
        cum = jnp.int32(0)
        found = jnp.int32(0)
        gstar = jnp.int32(0)
        need_g = jnp.int32(0)
        cs_g = zeros
        for g in range(NGRP - 1, -1, -1):
            hit = jnp.logical_and(found == 0, cum + gts[g] >= kin)
            hb = (zeros + hit.astype(jnp.int32)) == 1
            gstar = jnp.where(hit, g, gstar)
            need_g = jnp.where(hit, kin - cum, need_g)
            cs_g = jnp.where(hb, csg[g], cs_g)
            found = jnp.where(hit, 1, found)
            cum = cum + gts[g]
        q1 = jnp.max(plsc.all_reduce_ffs(cs_g >= need_g))
        tstar = gstar * L + (L - 1) - q1
        need_c = need_g - extract(cs_g, q1 - 1)
        cs_star = scr_v[pl.ds(tstar * L, L)]
        q2 = jnp.max(plsc.all_reduce_ffs(cs_star >= need_c))
        digit = tstar * L + (L - 1) - q2
        above = extract(cs_star, q2 - 1)
        kin_next = need_c - above
        cnt_b = extract(cs_star, q2) - above
        return digit, kin_next, cnt_b

    def compute_threshold(row_v, run_pass1):
        kin = jnp.int32(K)
        zero_hist()
        run_pass1()
        p1, kin, _ = scan_hist(kin)
        zero_hist()
        hist_pass2(row_v, p1)
        d2, kin, _ = scan_hist(kin)
        p12 = lax.bitwise_or(lax.shift_left(p1, 11), d2)
        zero_hist()
        hist_pass3(row_v, p12)
        d3, kin3, cnt3 = scan_hist(kin)
        T = lax.bitwise_or(lax.shift_left(p12, 9), d3)
        return T, cnt3 - kin3

    def output_pass(row_v, T):
        def obody(i, _):
            vs = [row_v[pl.ds((i * U + u) * L, L)] for u in range(U)]
            aa = [lax.bitwise_and(v, mask31) for v in vs]
            oo = [jnp.where(a >= T, a, 0) for a in aa]
            for u in range(U):
                row_v[pl.ds((i * U + u) * L, L)] = oo[u]
            return 0
        lax.fori_loop(0, NVEC // U, obody, 0)

    def in_copy(j, buf):
        return pltpu.make_async_copy(x_hbm.at[base + j], buf, isems[j % 3])

    def out_copy(j, buf):
        return pltpu.make_async_copy(buf, out_hbm.at[base + j], osems[j % 3])

    HV = NVEC // 2
    HN = N // 2

    def in_half(h):
        sem = isems[0] if h == 0 else si3
        return pltpu.make_async_copy(
            x_hbm.at[base, pl.ds(h * HN, HN)],
            bufs[0].at[pl.ds(h * HN, HN)], sem)

    in_half(0).start()
    in_half(1).start()
    for m in (1, 2):
        in_copy(m, bufs[m]).start()

    def fix_ties(row_v, T, extra):
        def cond(c):
            i, z = c
            return jnp.logical_and(i < NVEC // U, z < extra)

        def fbody(c):
            i, z = c
            os = [row_v[pl.ds((NVEC - (i * U + u + 1)) * L, L)]
                  for u in range(U)]
            mis = [(o == T).astype(jnp.int32) for o in os]
            rcss = [lax.rev(plsc.cumsum(lax.rev(mi, (0,))), (0,))
                    for mi in mis]
            ts = [jnp.max(rcs) for rcs in rcss]
            zs = [z]
            for u in range(1, U):
                zs.append(zs[u - 1] + ts[u - 1])
            for u in range(U):
                zm = jnp.logical_and(mis[u] == 1, rcss[u] + zs[u] <= extra)
                row_v[pl.ds((NVEC - (i * U + u + 1)) * L, L)] = (
                    jnp.where(zm, 0, os[u]))
            return (i + 1, zs[U - 1] + ts[U - 1])
        lax.while_loop(cond, fbody, (jnp.int32(0), jnp.int32(0)))

    for j in range(ROWS_PER_W):
        bj = bufs[j % 3]
        if j == 0:
            def run_pass1(b=bj):
                in_half(0).wait()
                hist_pass1(b, 0, HV)
                in_half(1).wait()
                hist_pass1(b, HV, HV)
        else:
            in_copy(j, bj).wait()

            def run_pass1(b=bj):
                hist_pass1(b)
        T, extra = compute_threshold(bj, run_pass1)
        if j == 1:
            out_copy(0, bufs[0]).wait()
            in_copy(3, bufs[0]).start()
        output_pass(bj, T)

        @pl.when(extra > 0)
        def _():
            fix_ties(bj, T, extra)
        out_copy(j, bj).start()

    for j in (1, 2, 3):
        out_copy(j, bufs[j % 3]).wait()


@jax.jit
def _topk_mask(bits):
    mesh = plsc.VectorSubcoreMesh(core_axis_name="c", subcore_axis_name="s")
    f = functools.partial(
        pl.kernel,
        out_type=jax.ShapeDtypeStruct((B, N), jnp.int32),
        mesh=mesh,
        scratch_types=[
            pltpu.VMEM((N,), jnp.int32),
            pltpu.VMEM((N,), jnp.int32),
            pltpu.VMEM((N,), jnp.int32),
            pltpu.VMEM((NB,), jnp.int32),
            pltpu.VMEM((NB,), jnp.int32),
            pltpu.SemaphoreType.DMA,
            pltpu.SemaphoreType.DMA,
            pltpu.SemaphoreType.DMA,
            pltpu.SemaphoreType.DMA,
            pltpu.SemaphoreType.DMA,
            pltpu.SemaphoreType.DMA,
            pltpu.SemaphoreType.DMA,
        ],
        compiler_params=pltpu.CompilerParams(needs_layout_passes=False),
    )(_topk_body)
    return f(bits)


def kernel(inputs):
    bits = lax.bitcast_convert_type(inputs, jnp.int32)
    out_bits = _topk_mask(bits)
    return lax.bitcast_convert_type(out_bits, jnp.float32)
